# baseline (device time: 58367 ns/iter reference)
import jax
import jax.numpy as jnp
from jax import lax
from jax.experimental import pallas as pl
from jax.experimental.pallas import tpu as pltpu

N_Z = 4


def kernel(Q, K, V):
    b, s, h, d = Q.shape
    bh = b * h
    scale = d ** -0.5

    def to3(a):
        return jnp.transpose(a, (0, 2, 1, 3)).reshape(bh, s, d).astype(jnp.bfloat16)

    Qr, Kr, Vr = to3(Q), to3(K), to3(V)

    def body(q_ref, k_ref, v_ref, out_ref, kbuf, vbuf,
             ksend, krecv, vsend, vrecv):
        my_x = lax.axis_index("x")
        my_y = lax.axis_index("y")
        my_z = lax.axis_index("z")
        right = lax.rem(my_z + 1, N_Z)
        left = lax.rem(my_z + N_Z - 1, N_Z)

        barrier = pltpu.get_barrier_semaphore()
        for nbr in (left, right):
            pl.semaphore_signal(barrier, inc=1,
                                device_id=(my_x, my_y, nbr),
                                device_id_type=pl.DeviceIdType.MESH)
        pl.semaphore_wait(barrier, 2)

        kbuf[0] = k_ref[...]
        vbuf[0] = v_ref[...]

        for hop in range(N_Z - 1):
            krd = pltpu.make_async_remote_copy(
                src_ref=kbuf.at[hop], dst_ref=kbuf.at[hop + 1],
                send_sem=ksend.at[hop], recv_sem=krecv.at[hop + 1],
                device_id=(my_x, my_y, right),
                device_id_type=pl.DeviceIdType.MESH)
            vrd = pltpu.make_async_remote_copy(
                src_ref=vbuf.at[hop], dst_ref=vbuf.at[hop + 1],
                send_sem=vsend.at[hop], recv_sem=vrecv.at[hop + 1],
                device_id=(my_x, my_y, right),
                device_id_type=pl.DeviceIdType.MESH)
            del vrd
            krd.start()
            krd.wait()

        for i in range(bh):
            q = q_ref[i]
            s_parts = [
                lax.dot_general(
                    q, kbuf[a, i],
                    dimension_numbers=(((1,), (1,)), ((), ())),
                    preferred_element_type=jnp.float32)
                for a in range(N_Z)
            ]
            sc = jnp.concatenate(s_parts, axis=1) * scale
            m = jnp.max(sc, axis=1, keepdims=True)
            p = jnp.exp(sc - m)
            l = jnp.sum(p, axis=1, keepdims=True)
            p = (p / l).astype(jnp.bfloat16)
            acc = jnp.zeros((s, d), jnp.float32)
            for a in range(N_Z):
                acc += lax.dot_general(
                    p[:, a * s:(a + 1) * s], vbuf[a, i],
                    dimension_numbers=(((1,), (0,)), ((), ())),
                    preferred_element_type=jnp.float32)
            out_ref[i] = acc

    out = pl.pallas_call(
        body,
        out_shape=jax.ShapeDtypeStruct((bh, s, d), jnp.float32),
        in_specs=[pl.BlockSpec(memory_space=pltpu.VMEM)] * 3,
        out_specs=pl.BlockSpec(memory_space=pltpu.VMEM),
        scratch_shapes=[
            pltpu.VMEM((N_Z, bh, s, d), jnp.bfloat16),
            pltpu.VMEM((N_Z, bh, s, d), jnp.bfloat16),
            pltpu.SemaphoreType.DMA((N_Z,)),
            pltpu.SemaphoreType.DMA((N_Z,)),
            pltpu.SemaphoreType.DMA((N_Z,)),
            pltpu.SemaphoreType.DMA((N_Z,)),
        ],
        compiler_params=pltpu.CompilerParams(collective_id=0),
    )(Qr, Kr, Vr)

    return jnp.transpose(out.reshape(b, h, s, d), (0, 2, 1, 3))


# device time: 53186 ns/iter; 1.0974x vs baseline; 1.0974x over previous
import jax
import jax.numpy as jnp
from jax import lax
from jax.experimental import pallas as pl
from jax.experimental.pallas import tpu as pltpu

N_Z = 4
N_ROLE = 4
MESH = pl.DeviceIdType.MESH


def kernel(Q, K, V):
    b, s, h, d = Q.shape
    bh = b * h
    hpr = bh // N_ROLE
    scale = d ** -0.5

    def heads3(a):
        return jnp.transpose(a, (0, 2, 1, 3)).reshape(bh, s, d).astype(jnp.bfloat16)

    Qr = heads3(Q)
    Kq = heads3(K).reshape(N_ROLE, hpr, s, d)
    Vq = heads3(V).reshape(N_ROLE, hpr, s, d)
    KVr = jnp.stack([Kq, Vq], axis=1)

    def flash(st, q, k, v):
        sc = lax.dot_general(q, k, (((1,), (1,)), ((), ())),
                             preferred_element_type=jnp.float32) * scale
        mc = jnp.max(sc, axis=1, keepdims=True)
        if st is None:
            m = mc
            p = jnp.exp(sc - m)
            l = jnp.sum(p, axis=1, keepdims=True)
            acc = lax.dot_general(p.astype(jnp.bfloat16), v,
                                  (((1,), (0,)), ((), ())),
                                  preferred_element_type=jnp.float32)
            return (m, l, acc)
        m0, l0, a0 = st
        m = jnp.maximum(m0, mc)
        corr = jnp.exp(m0 - m)
        p = jnp.exp(sc - m)
        l = l0 * corr + jnp.sum(p, axis=1, keepdims=True)
        acc = a0 * corr + lax.dot_general(p.astype(jnp.bfloat16), v,
                                          (((1,), (0,)), ((), ())),
                                          preferred_element_type=jnp.float32)
        return (m, l, acc)

    def body(q_ref, kv_ref, out_ref, kvbuf,
             zsend, zrecv, sx1, rx1, sy1, ry1, sx2, rx2, sy2, ry2):
        my_x = lax.axis_index("x")
        my_y = lax.axis_index("y")
        my_z = lax.axis_index("z")
        r_me = my_x * 2 + my_y
        r_x = (1 - my_x) * 2 + my_y
        r_y = my_x * 2 + (1 - my_y)
        xn = (1 - my_x, my_y, my_z)
        yn = (my_x, 1 - my_y, my_z)

        barrier = pltpu.get_barrier_semaphore()
        for dz in (1, 2, 3):
            pl.semaphore_signal(barrier, inc=1,
                                device_id=(my_x, my_y, lax.rem(my_z + dz, N_Z)),
                                device_id_type=MESH)
        pl.semaphore_signal(barrier, inc=1, device_id=xn, device_id_type=MESH)
        pl.semaphore_signal(barrier, inc=1, device_id=yn, device_id_type=MESH)
        pl.semaphore_wait(barrier, 5)

        pending = []

        for dz in (1, 2, 3):
            tz = lax.rem(my_z + dz, N_Z)
            zd = pltpu.make_async_remote_copy(
                src_ref=kv_ref.at[r_me],
                dst_ref=kvbuf.at[my_z, r_me],
                send_sem=zsend.at[dz],
                recv_sem=zrecv.at[N_Z - dz],
                device_id=(my_x, my_y, tz), device_id_type=MESH)
            zd.start()
            pending.append(zd)

        for rel in (1, 2, 3):
            o = lax.rem(my_z + rel, N_Z)
            rcv = pltpu.make_async_remote_copy(
                src_ref=kv_ref.at[r_me], dst_ref=kvbuf.at[o, r_me],
                send_sem=zsend.at[rel], recv_sem=zrecv.at[rel],
                device_id=(my_x, my_y, o), device_id_type=MESH)
            rcv.wait_recv()
            for nbr, ssem, rsem in ((xn, sx1, rx1), (yn, sy1, ry1)):
                fwd = pltpu.make_async_remote_copy(
                    src_ref=kvbuf.at[o, r_me], dst_ref=kvbuf.at[o, r_me],
                    send_sem=ssem.at[rel], recv_sem=rsem.at[rel],
                    device_id=nbr, device_id_type=MESH)
                fwd.start()
                pending.append(fwd)

        state = [None] * bh
        def own_quarter(j):
            for hh in range(hpr):
                i = j * hpr + hh
                state[i] = flash(state[i], q_ref[i],
                                 kv_ref[j, 0, hh], kv_ref[j, 1, hh])

        for rel in (1, 2, 3):
            o = lax.rem(my_z + rel, N_Z)
            rcvx = pltpu.make_async_remote_copy(
                src_ref=kvbuf.at[o, r_x], dst_ref=kvbuf.at[o, r_x],
                send_sem=sx1.at[rel], recv_sem=rx1.at[rel],
                device_id=xn, device_id_type=MESH)
            rcvx.wait_recv()
            f_v = pltpu.make_async_remote_copy(
                src_ref=kvbuf.at[o, r_x, 1], dst_ref=kvbuf.at[o, r_x, 1],
                send_sem=sy2.at[rel], recv_sem=ry2.at[rel],
                device_id=yn, device_id_type=MESH)
            f_v.start()
            pending.append(f_v)
            rcvy = pltpu.make_async_remote_copy(
                src_ref=kvbuf.at[o, r_y], dst_ref=kvbuf.at[o, r_y],
                send_sem=sy1.at[rel], recv_sem=ry1.at[rel],
                device_id=yn, device_id_type=MESH)
            rcvy.wait_recv()
            f_k = pltpu.make_async_remote_copy(
                src_ref=kvbuf.at[o, r_y, 0], dst_ref=kvbuf.at[o, r_y, 0],
                send_sem=sx2.at[rel], recv_sem=rx2.at[rel],
                device_id=xn, device_id_type=MESH)
            f_k.start()
            pending.append(f_k)
            own_quarter(rel - 1)
        own_quarter(N_ROLE - 1)

        for rel in (1, 2, 3):
            o = lax.rem(my_z + rel, N_Z)
            for rsem, kvhalf in ((rx2, 0), (ry2, 1)):
                w = pltpu.make_async_remote_copy(
                    src_ref=kvbuf.at[o, r_me, kvhalf],
                    dst_ref=kvbuf.at[o, r_me, kvhalf],
                    send_sem=sx2.at[rel], recv_sem=rsem.at[rel],
                    device_id=xn, device_id_type=MESH)
                w.wait_recv()
            for j in range(N_ROLE):
                for hh in range(hpr):
                    i = j * hpr + hh
                    state[i] = flash(state[i], q_ref[i],
                                     kvbuf[o, j, 0, hh], kvbuf[o, j, 1, hh])

        for i in range(bh):
            m, l, acc = state[i]
            out_ref[i] = acc / l

        for dsc in pending:
            dsc.wait_send()

    out = pl.pallas_call(
        body,
        out_shape=jax.ShapeDtypeStruct((bh, s, d), jnp.float32),
        in_specs=[pl.BlockSpec(memory_space=pltpu.VMEM)] * 2,
        out_specs=pl.BlockSpec(memory_space=pltpu.VMEM),
        scratch_shapes=[
            pltpu.VMEM((N_Z, N_ROLE, 2, hpr, s, d), jnp.bfloat16),
        ] + [pltpu.SemaphoreType.DMA((N_Z,)) for _ in range(10)],
        compiler_params=pltpu.CompilerParams(collective_id=0),
    )(Qr, KVr)

    return jnp.transpose(out.reshape(b, h, s, d), (0, 2, 1, 3))
